# Initial kernel scaffold; baseline (speedup 1.0000x reference)
import jax
import jax.numpy as jnp
from jax.experimental import pallas as pl


def kernel(h):
    raise NotImplementedError("write your pallas kernel here")


# speedup vs baseline: 6.2769x; 6.2769x over previous
import jax
import jax.numpy as jnp
from jax.experimental import pallas as pl

NEIGHBORS = 16
ROWS = 128


def _knn_kernel(x_rows_ref, x_all_ref, dst_ref):
    s = pl.program_id(0)
    x_all = x_all_ref[0]
    x_rows = x_rows_ref[0]
    n_points = x_all.shape[0]
    x2 = jnp.sum(x_all * x_all, axis=1, keepdims=True)
    m = jax.lax.dot_general(
        x_all, x_rows, (((1,), (1,)), ((), ())),
        preferred_element_type=jnp.float32,
        precision=jax.lax.Precision.DEFAULT)
    cur = x2 - 2.0 * m
    cidx = jax.lax.broadcasted_iota(jnp.int32, (n_points, ROWS), 0)
    rows = []
    for _ in range(NEIGHBORS):
        mn = jnp.min(cur, axis=0, keepdims=True)
        cand = jnp.where(cur <= mn, cidx, n_points)
        idx = jnp.min(cand, axis=0, keepdims=True)
        rows.append(idx)
        cur = jnp.where(cidx == idx, jnp.inf, cur)
    blk = jnp.concatenate(rows, axis=0)
    dst_ref[0] = blk + s * n_points


def kernel(h):
    n_samples, n_points, n_dims = h.shape
    grid = (n_samples, n_points // ROWS)
    dst_t = pl.pallas_call(
        _knn_kernel,
        grid=grid,
        in_specs=[
            pl.BlockSpec((1, ROWS, n_dims), lambda s, rb: (s, rb, 0)),
            pl.BlockSpec((1, n_points, n_dims), lambda s, rb: (s, 0, 0)),
        ],
        out_specs=pl.BlockSpec((1, NEIGHBORS, ROWS), lambda s, rb: (s, 0, rb)),
        out_shape=jax.ShapeDtypeStruct((n_samples, NEIGHBORS, n_points),
                                       jnp.int32),
    )(h, h)
    dst = jnp.swapaxes(dst_t, 1, 2).reshape(-1)
    src = (jnp.arange(n_points, dtype=jnp.int32)[None, :, None]
           + (jnp.arange(n_samples, dtype=jnp.int32) * n_points)[:, None,
                                                                 None])
    src = jnp.broadcast_to(src,
                           (n_samples, n_points, NEIGHBORS)).reshape(-1)
    return jnp.stack([src, dst], axis=0)


# speedup vs baseline: 26.0924x; 4.1569x over previous
import jax
import jax.numpy as jnp
from jax.experimental import pallas as pl
from jax.experimental.pallas import tpu as pltpu

NEIGHBORS = 16
ROWS = 128


def _oddeven_merge_sort_pairs(n):
    pairs = []
    p = 1
    while p < n:
        k = p
        while k >= 1:
            for j in range(k % p, n - k, 2 * k):
                for i in range(0, min(k, n - j - k)):
                    if (i + j) // (2 * p) == (i + j + k) // (2 * p):
                        pairs.append((i + j, i + j + k))
            k //= 2
        p *= 2
    return pairs


_PAIRS16 = _oddeven_merge_sort_pairs(16)


def _ce(vs, ii, i, j):
    a, b = vs[i], vs[j]
    sw = a > b
    vs[i] = jnp.minimum(a, b)
    vs[j] = jnp.maximum(a, b)
    ia, ib = ii[i], ii[j]
    ii[i] = jnp.where(sw, ib, ia)
    ii[j] = jnp.where(sw, ia, ib)


def _merge_keep16(vsA, iiA, vsB, iiB):
    vs, ii = [], []
    for u in range(NEIGHBORS):
        a, b = vsA[u], vsB[15 - u]
        sw = a > b
        vs.append(jnp.minimum(a, b))
        ii.append(jnp.where(sw, iiB[15 - u], iiA[u]))
    for dist in (8, 4, 2, 1):
        for u in range(NEIGHBORS):
            if u % (2 * dist) < dist:
                _ce(vs, ii, u, u + dist)
    return vs, ii


def _knn_kernel(x_rows_ref, x_all_ref, dst_ref):
    s = pl.program_id(0)
    x_all = x_all_ref[0]
    x_rows = x_rows_ref[0]
    n_points = x_all.shape[0]
    x2 = jnp.sum(x_all * x_all, axis=1, keepdims=True)
    m = jax.lax.dot_general(
        x_all, x_rows, (((1,), (1,)), ((), ())),
        preferred_element_type=jnp.float32,
        precision=jax.lax.Precision.DEFAULT)
    d = x2 - 2.0 * m
    cidx = jax.lax.broadcasted_iota(jnp.int32, (n_points, ROWS), 0)

    n_groups = n_points // 128
    d4 = d.reshape(n_groups, 16, 8, ROWS)
    i4 = cidx.reshape(n_groups, 16, 8, ROWS)
    vs = [d4[:, u] for u in range(16)]
    ii = [i4[:, u] for u in range(16)]
    for i, j in _PAIRS16:
        _ce(vs, ii, i, j)

    g = n_groups
    while g > 1:
        vsA = [v.reshape(g // 2, 2, 8, ROWS)[:, 0] for v in vs]
        vsB = [v.reshape(g // 2, 2, 8, ROWS)[:, 1] for v in vs]
        iiA = [v.reshape(g // 2, 2, 8, ROWS)[:, 0] for v in ii]
        iiB = [v.reshape(g // 2, 2, 8, ROWS)[:, 1] for v in ii]
        vs, ii = _merge_keep16(vsA, iiA, vsB, iiB)
        g //= 2

    for k in (4, 2, 1):
        vsB = [pltpu.roll(v, 8 - k, 1) for v in vs]
        iiB = [pltpu.roll(v, 8 - k, 1) for v in ii]
        vs, ii = _merge_keep16(vs, ii, vsB, iiB)

    siota = jax.lax.broadcasted_iota(jnp.int32, (1, 8, ROWS), 1)
    offset = s * n_points
    packed = []
    for half in range(2):
        p = jnp.zeros((1, 8, ROWS), jnp.int32)
        for u in range(8):
            rolled = pltpu.roll(ii[half * 8 + u], u, 1)
            p = jnp.where(siota == u, rolled, p)
        packed.append(p[0] + offset)
    dst_ref[0] = jnp.concatenate(packed, axis=0)


def kernel(h):
    n_samples, n_points, n_dims = h.shape
    grid = (n_samples, n_points // ROWS)
    dst_t = pl.pallas_call(
        _knn_kernel,
        grid=grid,
        in_specs=[
            pl.BlockSpec((1, ROWS, n_dims), lambda s, rb: (s, rb, 0)),
            pl.BlockSpec((1, n_points, n_dims), lambda s, rb: (s, 0, 0)),
        ],
        out_specs=pl.BlockSpec((1, NEIGHBORS, ROWS), lambda s, rb: (s, 0, rb)),
        out_shape=jax.ShapeDtypeStruct((n_samples, NEIGHBORS, n_points),
                                       jnp.int32),
    )(h, h)
    dst = jnp.swapaxes(dst_t, 1, 2).reshape(-1)
    src = (jnp.arange(n_points, dtype=jnp.int32)[None, :, None]
           + (jnp.arange(n_samples, dtype=jnp.int32) * n_points)[:, None,
                                                                 None])
    src = jnp.broadcast_to(src,
                           (n_samples, n_points, NEIGHBORS)).reshape(-1)
    return jnp.stack([src, dst], axis=0)


# speedup vs baseline: 29.6092x; 1.1348x over previous
import jax
import jax.numpy as jnp
from jax.experimental import pallas as pl
from jax.experimental.pallas import tpu as pltpu

NEIGHBORS = 16
ROWS = 128


def _oddeven_merge_sort_pairs(n):
    pairs = []
    p = 1
    while p < n:
        k = p
        while k >= 1:
            for j in range(k % p, n - k, 2 * k):
                for i in range(0, min(k, n - j - k)):
                    if (i + j) // (2 * p) == (i + j + k) // (2 * p):
                        pairs.append((i + j, i + j + k))
            k //= 2
        p *= 2
    return pairs


_PAIRS16 = _oddeven_merge_sort_pairs(16)


def _ce(vs, ii, i, j):
    a, b = vs[i], vs[j]
    sw = a > b
    vs[i] = jnp.minimum(a, b)
    vs[j] = jnp.maximum(a, b)
    ia, ib = ii[i], ii[j]
    ii[i] = jnp.where(sw, ib, ia)
    ii[j] = jnp.where(sw, ia, ib)


def _merge_keep16(vsA, iiA, vsB, iiB):
    vs, ii = [], []
    for u in range(NEIGHBORS):
        a, b = vsA[u], vsB[15 - u]
        sw = a > b
        vs.append(jnp.minimum(a, b))
        ii.append(jnp.where(sw, iiB[15 - u], iiA[u]))
    for dist in (8, 4, 2, 1):
        for u in range(NEIGHBORS):
            if u % (2 * dist) < dist:
                _ce(vs, ii, u, u + dist)
    return vs, ii


def _knn_kernel(x_rows_ref, x_all_ref, dst_ref, x2_ref):
    s = pl.program_id(0)
    rb = pl.program_id(1)
    x_all = x_all_ref[0]
    x_rows = x_rows_ref[0]
    n_points = x_all.shape[0]

    @pl.when(rb == 0)
    def _():
        x2 = jnp.sum(x_all * x_all, axis=1, keepdims=True)
        x2_ref[...] = jnp.broadcast_to(x2, (n_points, ROWS))

    m = jax.lax.dot_general(
        x_all, -2.0 * x_rows, (((1,), (1,)), ((), ())),
        preferred_element_type=jnp.float32,
        precision=jax.lax.Precision.DEFAULT)
    d = x2_ref[...] + m

    n_groups = n_points // 128
    d4 = d.reshape(n_groups, 16, 8, ROWS)
    vs = []
    for u in range(16):
        ki = jax.lax.bitcast_convert_type(d4[:, u], jnp.int32)
        vs.append(jax.lax.bitcast_convert_type((ki & -16) | u, jnp.float32))
    for i, j in _PAIRS16:
        a, b = vs[i], vs[j]
        vs[i] = jnp.minimum(a, b)
        vs[j] = jnp.maximum(a, b)
    giota = jax.lax.broadcasted_iota(jnp.int32, (n_groups, 8, ROWS), 0)
    siota = jax.lax.broadcasted_iota(jnp.int32, (n_groups, 8, ROWS), 1)
    base = (giota << 7) + siota
    ii = []
    for u in range(16):
        ki = jax.lax.bitcast_convert_type(vs[u], jnp.int32)
        ii.append(base + ((ki & 15) << 3))

    g = n_groups
    while g > 1:
        vsA = [v.reshape(g // 2, 2, 8, ROWS)[:, 0] for v in vs]
        vsB = [v.reshape(g // 2, 2, 8, ROWS)[:, 1] for v in vs]
        iiA = [v.reshape(g // 2, 2, 8, ROWS)[:, 0] for v in ii]
        iiB = [v.reshape(g // 2, 2, 8, ROWS)[:, 1] for v in ii]
        vs, ii = _merge_keep16(vsA, iiA, vsB, iiB)
        g //= 2

    for k in (4, 2, 1):
        vsB = [pltpu.roll(v, 8 - k, 1) for v in vs]
        iiB = [pltpu.roll(v, 8 - k, 1) for v in ii]
        vs, ii = _merge_keep16(vs, ii, vsB, iiB)

    sl = jax.lax.broadcasted_iota(jnp.int32, (1, 8, ROWS), 1)
    offset = s * n_points
    packed = []
    for half in range(2):
        p = jnp.zeros((1, 8, ROWS), jnp.int32)
        for u in range(8):
            rolled = pltpu.roll(ii[half * 8 + u], u, 1)
            p = jnp.where(sl == u, rolled, p)
        packed.append(p[0] + offset)
    dst_ref[0] = jnp.concatenate(packed, axis=0)


def kernel(h):
    n_samples, n_points, n_dims = h.shape
    grid = (n_samples, n_points // ROWS)
    dst_t = pl.pallas_call(
        _knn_kernel,
        grid=grid,
        in_specs=[
            pl.BlockSpec((1, ROWS, n_dims), lambda s, rb: (s, rb, 0)),
            pl.BlockSpec((1, n_points, n_dims), lambda s, rb: (s, 0, 0)),
        ],
        out_specs=pl.BlockSpec((1, NEIGHBORS, ROWS), lambda s, rb: (s, 0, rb)),
        out_shape=jax.ShapeDtypeStruct((n_samples, NEIGHBORS, n_points),
                                       jnp.int32),
        scratch_shapes=[pltpu.VMEM((n_points, ROWS), jnp.float32)],
    )(h, h)
    dst = jnp.swapaxes(dst_t, 1, 2).reshape(-1)
    src = (jnp.arange(n_points, dtype=jnp.int32)[None, :, None]
           + (jnp.arange(n_samples, dtype=jnp.int32) * n_points)[:, None,
                                                                 None])
    src = jnp.broadcast_to(src,
                           (n_samples, n_points, NEIGHBORS)).reshape(-1)
    return jnp.stack([src, dst], axis=0)


# speedup vs baseline: 37.2666x; 1.2586x over previous
import jax
import jax.numpy as jnp
from jax.experimental import pallas as pl
from jax.experimental.pallas import tpu as pltpu

NEIGHBORS = 16
ROWS = 128


def _oddeven_merge_sort_pairs(n):
    pairs = []
    p = 1
    while p < n:
        k = p
        while k >= 1:
            for j in range(k % p, n - k, 2 * k):
                for i in range(0, min(k, n - j - k)):
                    if (i + j) // (2 * p) == (i + j + k) // (2 * p):
                        pairs.append((i + j, i + j + k))
            k //= 2
        p *= 2
    return pairs


_PAIRS16 = _oddeven_merge_sort_pairs(16)


def _merge_keep16(vsA, vsB):
    vs = [jnp.minimum(vsA[u], vsB[15 - u]) for u in range(NEIGHBORS)]
    for dist in (8, 4, 2, 1):
        for u in range(NEIGHBORS):
            if u % (2 * dist) < dist:
                a, b = vs[u], vs[u + dist]
                vs[u] = jnp.minimum(a, b)
                vs[u + dist] = jnp.maximum(a, b)
    return vs


def _merge_keep16_idx(vsA, iiA, vsB, iiB):
    vs, ii = [], []
    for u in range(NEIGHBORS):
        a, b = vsA[u], vsB[15 - u]
        sw = a > b
        vs.append(jnp.minimum(a, b))
        ii.append(jnp.where(sw, iiB[15 - u], iiA[u]))
    for dist in (8, 4, 2, 1):
        for u in range(NEIGHBORS):
            if u % (2 * dist) < dist:
                a, b = vs[u], vs[u + dist]
                sw = a > b
                vs[u] = jnp.minimum(a, b)
                vs[u + dist] = jnp.maximum(a, b)
                ia, ib = ii[u], ii[u + dist]
                ii[u] = jnp.where(sw, ib, ia)
                ii[u + dist] = jnp.where(sw, ia, ib)
    return vs, ii


def _knn_kernel(x_rows_ref, x_all_ref, dst_ref, x2_ref):
    s = pl.program_id(0)
    rb = pl.program_id(1)
    x_all = x_all_ref[0]
    x_rows = x_rows_ref[0]
    n_points = x_all.shape[0]

    @pl.when(rb == 0)
    def _():
        x2 = jnp.sum(x_all * x_all, axis=1, keepdims=True)
        x2_ref[...] = jnp.broadcast_to(x2, (n_points, ROWS))

    m = jax.lax.dot_general(
        x_all, -2.0 * x_rows, (((1,), (1,)), ((), ())),
        preferred_element_type=jnp.float32,
        precision=jax.lax.Precision.DEFAULT)
    d = x2_ref[...] + m

    n_groups = n_points // 128
    d4 = d.reshape(n_groups, 16, 8, ROWS)
    giota = jax.lax.broadcasted_iota(jnp.int32, (n_groups, 8, ROWS), 0)
    tcol = giota << 4
    vs = []
    for u in range(16):
        ki = jax.lax.bitcast_convert_type(d4[:, u], jnp.int32)
        vs.append(jax.lax.bitcast_convert_type((ki & -256) | (tcol | u),
                                               jnp.float32))
    for i, j in _PAIRS16:
        a, b = vs[i], vs[j]
        vs[i] = jnp.minimum(a, b)
        vs[j] = jnp.maximum(a, b)

    g = n_groups
    while g > 1:
        vsA = [v.reshape(g // 2, 2, 8, ROWS)[:, 0] for v in vs]
        vsB = [v.reshape(g // 2, 2, 8, ROWS)[:, 1] for v in vs]
        vs = _merge_keep16(vsA, vsB)
        g //= 2

    siota = jax.lax.broadcasted_iota(jnp.int32, (1, 8, ROWS), 1)
    ii = []
    for u in range(16):
        ki = jax.lax.bitcast_convert_type(vs[u], jnp.int32)
        ii.append(((ki & 255) << 3) | siota)

    for k in (4, 2, 1):
        vsB = [pltpu.roll(v, 8 - k, 1) for v in vs]
        iiB = [pltpu.roll(v, 8 - k, 1) for v in ii]
        vs, ii = _merge_keep16_idx(vs, ii, vsB, iiB)

    offset = s * n_points
    packed = []
    for half in range(2):
        p = jnp.zeros((1, 8, ROWS), jnp.int32)
        for u in range(8):
            rolled = pltpu.roll(ii[half * 8 + u], u, 1)
            p = jnp.where(siota == u, rolled, p)
        packed.append(p[0] + offset)
    dst_ref[0] = jnp.concatenate(packed, axis=0)


def kernel(h):
    n_samples, n_points, n_dims = h.shape
    grid = (n_samples, n_points // ROWS)
    dst_t = pl.pallas_call(
        _knn_kernel,
        grid=grid,
        in_specs=[
            pl.BlockSpec((1, ROWS, n_dims), lambda s, rb: (s, rb, 0)),
            pl.BlockSpec((1, n_points, n_dims), lambda s, rb: (s, 0, 0)),
        ],
        out_specs=pl.BlockSpec((1, NEIGHBORS, ROWS), lambda s, rb: (s, 0, rb)),
        out_shape=jax.ShapeDtypeStruct((n_samples, NEIGHBORS, n_points),
                                       jnp.int32),
        scratch_shapes=[pltpu.VMEM((n_points, ROWS), jnp.float32)],
    )(h, h)
    dst = jnp.swapaxes(dst_t, 1, 2).reshape(-1)
    src = (jnp.arange(n_points, dtype=jnp.int32)[None, :, None]
           + (jnp.arange(n_samples, dtype=jnp.int32) * n_points)[:, None,
                                                                 None])
    src = jnp.broadcast_to(src,
                           (n_samples, n_points, NEIGHBORS)).reshape(-1)
    return jnp.stack([src, dst], axis=0)


# speedup vs baseline: 45.7673x; 1.2281x over previous
import jax
import jax.numpy as jnp
from jax.experimental import pallas as pl
from jax.experimental.pallas import tpu as pltpu

NEIGHBORS = 16
ROWS = 512


def _oddeven_merge_sort_pairs(n):
    pairs = []
    p = 1
    while p < n:
        k = p
        while k >= 1:
            for j in range(k % p, n - k, 2 * k):
                for i in range(0, min(k, n - j - k)):
                    if (i + j) // (2 * p) == (i + j + k) // (2 * p):
                        pairs.append((i + j, i + j + k))
            k //= 2
        p *= 2
    return pairs


_PAIRS16 = _oddeven_merge_sort_pairs(16)


def _merge_keep16(vsA, vsB):
    vs = [jnp.minimum(vsA[u], vsB[15 - u]) for u in range(NEIGHBORS)]
    for dist in (8, 4, 2, 1):
        for u in range(NEIGHBORS):
            if u % (2 * dist) < dist:
                a, b = vs[u], vs[u + dist]
                vs[u] = jnp.minimum(a, b)
                vs[u + dist] = jnp.maximum(a, b)
    return vs


def _merge_keep16_idx(vsA, iiA, vsB, iiB):
    vs, ii = [], []
    for u in range(NEIGHBORS):
        a, b = vsA[u], vsB[15 - u]
        sw = a > b
        vs.append(jnp.minimum(a, b))
        ii.append(jnp.where(sw, iiB[15 - u], iiA[u]))
    for dist in (8, 4, 2, 1):
        for u in range(NEIGHBORS):
            if u % (2 * dist) < dist:
                a, b = vs[u], vs[u + dist]
                sw = a > b
                vs[u] = jnp.minimum(a, b)
                vs[u + dist] = jnp.maximum(a, b)
                ia, ib = ii[u], ii[u + dist]
                ii[u] = jnp.where(sw, ib, ia)
                ii[u + dist] = jnp.where(sw, ia, ib)
    return vs, ii


def _knn_kernel(x_rows_ref, x_all_ref, dst_ref, x2_ref):
    s = pl.program_id(0)
    rb = pl.program_id(1)
    x_all = x_all_ref[0]
    x_rows = x_rows_ref[0]
    n_points = x_all.shape[0]

    @pl.when(rb == 0)
    def _():
        x2 = jnp.sum(x_all * x_all, axis=1, keepdims=True)
        x2_ref[...] = jnp.broadcast_to(x2, (n_points, ROWS))

    m = jax.lax.dot_general(
        x_all, -2.0 * x_rows, (((1,), (1,)), ((), ())),
        preferred_element_type=jnp.float32,
        precision=jax.lax.Precision.DEFAULT)
    d = x2_ref[...] + m

    n_groups = n_points // 128
    d4 = d.reshape(n_groups, 16, 8, ROWS)
    giota = jax.lax.broadcasted_iota(jnp.int32, (n_groups, 8, ROWS), 0)
    tcol = giota << 4
    vs = []
    for u in range(16):
        ki = jax.lax.bitcast_convert_type(d4[:, u], jnp.int32)
        vs.append(jax.lax.bitcast_convert_type((ki & -256) | (tcol | u),
                                               jnp.float32))
    for i, j in _PAIRS16:
        a, b = vs[i], vs[j]
        vs[i] = jnp.minimum(a, b)
        vs[j] = jnp.maximum(a, b)

    g = n_groups
    while g > 1:
        vsA = [v.reshape(g // 2, 2, 8, ROWS)[:, 0] for v in vs]
        vsB = [v.reshape(g // 2, 2, 8, ROWS)[:, 1] for v in vs]
        vs = _merge_keep16(vsA, vsB)
        g //= 2

    siota = jax.lax.broadcasted_iota(jnp.int32, (1, 8, ROWS), 1)
    ii = []
    for u in range(16):
        ki = jax.lax.bitcast_convert_type(vs[u], jnp.int32)
        ii.append(((ki & 255) << 3) | siota)

    for k in (4, 2, 1):
        vsB = [pltpu.roll(v, 8 - k, 1) for v in vs]
        iiB = [pltpu.roll(v, 8 - k, 1) for v in ii]
        vs, ii = _merge_keep16_idx(vs, ii, vsB, iiB)

    offset = s * n_points
    packed = []
    for half in range(2):
        p = jnp.zeros((1, 8, ROWS), jnp.int32)
        for u in range(8):
            rolled = pltpu.roll(ii[half * 8 + u], u, 1)
            p = jnp.where(siota == u, rolled, p)
        packed.append(p[0] + offset)
    dst_ref[0] = jnp.concatenate(packed, axis=0)


def kernel(h):
    n_samples, n_points, n_dims = h.shape
    grid = (n_samples, n_points // ROWS)
    dst_t = pl.pallas_call(
        _knn_kernel,
        grid=grid,
        in_specs=[
            pl.BlockSpec((1, ROWS, n_dims), lambda s, rb: (s, rb, 0)),
            pl.BlockSpec((1, n_points, n_dims), lambda s, rb: (s, 0, 0)),
        ],
        out_specs=pl.BlockSpec((1, NEIGHBORS, ROWS), lambda s, rb: (s, 0, rb)),
        out_shape=jax.ShapeDtypeStruct((n_samples, NEIGHBORS, n_points),
                                       jnp.int32),
        scratch_shapes=[pltpu.VMEM((n_points, ROWS), jnp.float32)],
    )(h, h)
    dst = jnp.swapaxes(dst_t, 1, 2).reshape(-1)
    src = (jnp.arange(n_points, dtype=jnp.int32)[None, :, None]
           + (jnp.arange(n_samples, dtype=jnp.int32) * n_points)[:, None,
                                                                 None])
    src = jnp.broadcast_to(src,
                           (n_samples, n_points, NEIGHBORS)).reshape(-1)
    return jnp.stack([src, dst], axis=0)


# speedup vs baseline: 46.5319x; 1.0167x over previous
import jax
import jax.numpy as jnp
from jax.experimental import pallas as pl
from jax.experimental.pallas import tpu as pltpu

NEIGHBORS = 16
ROWS = 1024


def _oddeven_merge_sort_pairs(n):
    pairs = []
    p = 1
    while p < n:
        k = p
        while k >= 1:
            for j in range(k % p, n - k, 2 * k):
                for i in range(0, min(k, n - j - k)):
                    if (i + j) // (2 * p) == (i + j + k) // (2 * p):
                        pairs.append((i + j, i + j + k))
            k //= 2
        p *= 2
    return pairs


_PAIRS16 = _oddeven_merge_sort_pairs(16)


def _merge_keep16(vsA, vsB):
    vs = [jnp.minimum(vsA[u], vsB[15 - u]) for u in range(NEIGHBORS)]
    for dist in (8, 4, 2, 1):
        for u in range(NEIGHBORS):
            if u % (2 * dist) < dist:
                a, b = vs[u], vs[u + dist]
                vs[u] = jnp.minimum(a, b)
                vs[u + dist] = jnp.maximum(a, b)
    return vs


def _merge_keep16_idx(vsA, iiA, vsB, iiB):
    vs, ii = [], []
    for u in range(NEIGHBORS):
        a, b = vsA[u], vsB[15 - u]
        sw = a > b
        vs.append(jnp.minimum(a, b))
        ii.append(jnp.where(sw, iiB[15 - u], iiA[u]))
    for dist in (8, 4, 2, 1):
        for u in range(NEIGHBORS):
            if u % (2 * dist) < dist:
                a, b = vs[u], vs[u + dist]
                sw = a > b
                vs[u] = jnp.minimum(a, b)
                vs[u + dist] = jnp.maximum(a, b)
                ia, ib = ii[u], ii[u + dist]
                ii[u] = jnp.where(sw, ib, ia)
                ii[u + dist] = jnp.where(sw, ia, ib)
    return vs, ii


def _knn_kernel(x_rows_ref, x_all_ref, dst_ref, x2_ref):
    s = pl.program_id(0)
    rb = pl.program_id(1)
    x_all = x_all_ref[0]
    x_rows = x_rows_ref[0]
    n_points = x_all.shape[0]

    @pl.when(rb == 0)
    def _():
        x2 = jnp.sum(x_all * x_all, axis=1, keepdims=True)
        x2_ref[...] = jnp.broadcast_to(x2, (n_points, ROWS))

    m = jax.lax.dot_general(
        x_all, -2.0 * x_rows, (((1,), (1,)), ((), ())),
        preferred_element_type=jnp.float32,
        precision=jax.lax.Precision.DEFAULT)
    d = x2_ref[...] + m

    n_groups = n_points // 128
    d4 = d.reshape(n_groups, 16, 8, ROWS)
    giota = jax.lax.broadcasted_iota(jnp.int32, (n_groups, 8, ROWS), 0)
    tcol = giota << 4
    vs = []
    for u in range(16):
        ki = jax.lax.bitcast_convert_type(d4[:, u], jnp.int32)
        vs.append(jax.lax.bitcast_convert_type((ki & -256) | (tcol | u),
                                               jnp.float32))
    for i, j in _PAIRS16:
        a, b = vs[i], vs[j]
        vs[i] = jnp.minimum(a, b)
        vs[j] = jnp.maximum(a, b)

    g = n_groups
    while g > 1:
        vsA = [v.reshape(g // 2, 2, 8, ROWS)[:, 0] for v in vs]
        vsB = [v.reshape(g // 2, 2, 8, ROWS)[:, 1] for v in vs]
        vs = _merge_keep16(vsA, vsB)
        g //= 2

    siota = jax.lax.broadcasted_iota(jnp.int32, (1, 8, ROWS), 1)
    ii = []
    for u in range(16):
        ki = jax.lax.bitcast_convert_type(vs[u], jnp.int32)
        ii.append(((ki & 255) << 3) | siota)

    for k in (4, 2, 1):
        vsB = [pltpu.roll(v, 8 - k, 1) for v in vs]
        iiB = [pltpu.roll(v, 8 - k, 1) for v in ii]
        vs, ii = _merge_keep16_idx(vs, ii, vsB, iiB)

    offset = s * n_points
    packed = []
    for half in range(2):
        p = jnp.zeros((1, 8, ROWS), jnp.int32)
        for u in range(8):
            rolled = pltpu.roll(ii[half * 8 + u], u, 1)
            p = jnp.where(siota == u, rolled, p)
        packed.append(p[0] + offset)
    dst_ref[0] = jnp.concatenate(packed, axis=0)


def kernel(h):
    n_samples, n_points, n_dims = h.shape
    grid = (n_samples, n_points // ROWS)
    dst_t = pl.pallas_call(
        _knn_kernel,
        grid=grid,
        in_specs=[
            pl.BlockSpec((1, ROWS, n_dims), lambda s, rb: (s, rb, 0)),
            pl.BlockSpec((1, n_points, n_dims), lambda s, rb: (s, 0, 0)),
        ],
        out_specs=pl.BlockSpec((1, NEIGHBORS, ROWS), lambda s, rb: (s, 0, rb)),
        out_shape=jax.ShapeDtypeStruct((n_samples, NEIGHBORS, n_points),
                                       jnp.int32),
        scratch_shapes=[pltpu.VMEM((n_points, ROWS), jnp.float32)],
    )(h, h)
    dst = jnp.swapaxes(dst_t, 1, 2).reshape(-1)
    src = (jnp.arange(n_points, dtype=jnp.int32)[None, :, None]
           + (jnp.arange(n_samples, dtype=jnp.int32) * n_points)[:, None,
                                                                 None])
    src = jnp.broadcast_to(src,
                           (n_samples, n_points, NEIGHBORS)).reshape(-1)
    return jnp.stack([src, dst], axis=0)


# speedup vs baseline: 49.5355x; 1.0646x over previous
import jax
import jax.numpy as jnp
from jax.experimental import pallas as pl
from jax.experimental.pallas import tpu as pltpu

NEIGHBORS = 16
ROWS = 1024


def _oddeven_merge_sort_pairs(n):
    pairs = []
    p = 1
    while p < n:
        k = p
        while k >= 1:
            for j in range(k % p, n - k, 2 * k):
                for i in range(0, min(k, n - j - k)):
                    if (i + j) // (2 * p) == (i + j + k) // (2 * p):
                        pairs.append((i + j, i + j + k))
            k //= 2
        p *= 2
    return pairs


_PAIRS16 = _oddeven_merge_sort_pairs(16)


def _merge_keep16(vsA, vsB):
    vs = [jnp.minimum(vsA[u], vsB[15 - u]) for u in range(NEIGHBORS)]
    for dist in (8, 4, 2, 1):
        for u in range(NEIGHBORS):
            if u % (2 * dist) < dist:
                a, b = vs[u], vs[u + dist]
                vs[u] = jnp.minimum(a, b)
                vs[u + dist] = jnp.maximum(a, b)
    return vs


def _merge_keep16_idx(vsA, iiA, vsB, iiB):
    vs, ii = [], []
    for u in range(NEIGHBORS):
        a, b = vsA[u], vsB[15 - u]
        sw = a > b
        vs.append(jnp.minimum(a, b))
        ii.append(jnp.where(sw, iiB[15 - u], iiA[u]))
    for dist in (8, 4, 2, 1):
        for u in range(NEIGHBORS):
            if u % (2 * dist) < dist:
                a, b = vs[u], vs[u + dist]
                sw = a > b
                vs[u] = jnp.minimum(a, b)
                vs[u + dist] = jnp.maximum(a, b)
                ia, ib = ii[u], ii[u + dist]
                ii[u] = jnp.where(sw, ib, ia)
                ii[u + dist] = jnp.where(sw, ia, ib)
    return vs, ii


def _knn_kernel(x_rows_ref, x_all_ref, out_ref, x2_ref):
    s = pl.program_id(0)
    rb = pl.program_id(1)
    x_all = x_all_ref[0]
    x_rows = x_rows_ref[0]
    n_points = x_all.shape[0]

    @pl.when(rb == 0)
    def _():
        x2 = jnp.sum(x_all * x_all, axis=1, keepdims=True)
        x2_ref[...] = jnp.broadcast_to(x2, (n_points, ROWS))

    m = jax.lax.dot_general(
        x_all, -2.0 * x_rows, (((1,), (1,)), ((), ())),
        preferred_element_type=jnp.float32,
        precision=jax.lax.Precision.DEFAULT)
    d = x2_ref[...] + m

    n_groups = n_points // 128
    d4 = d.reshape(n_groups, 16, 8, ROWS)
    giota = jax.lax.broadcasted_iota(jnp.int32, (n_groups, 8, ROWS), 0)
    tcol = giota << 4
    vs = []
    for u in range(16):
        ki = jax.lax.bitcast_convert_type(d4[:, u], jnp.int32)
        vs.append(jax.lax.bitcast_convert_type((ki & -256) | (tcol | u),
                                               jnp.float32))
    for i, j in _PAIRS16:
        a, b = vs[i], vs[j]
        vs[i] = jnp.minimum(a, b)
        vs[j] = jnp.maximum(a, b)

    g = n_groups
    while g > 1:
        vsA = [v.reshape(g // 2, 2, 8, ROWS)[:, 0] for v in vs]
        vsB = [v.reshape(g // 2, 2, 8, ROWS)[:, 1] for v in vs]
        vs = _merge_keep16(vsA, vsB)
        g //= 2

    siota = jax.lax.broadcasted_iota(jnp.int32, (1, 8, ROWS), 1)
    ii = []
    for u in range(16):
        ki = jax.lax.bitcast_convert_type(vs[u], jnp.int32)
        ii.append(((ki & 255) << 3) | siota)

    for k in (4, 2, 1):
        vsB = [pltpu.roll(v, 8 - k, 1) for v in vs]
        iiB = [pltpu.roll(v, 8 - k, 1) for v in ii]
        vs, ii = _merge_keep16_idx(vs, ii, vsB, iiB)

    offset = s * n_points
    packed = []
    for half in range(2):
        p = jnp.zeros((1, 8, ROWS), jnp.int32)
        for u in range(8):
            rolled = pltpu.roll(ii[half * 8 + u], u, 1)
            p = jnp.where(siota == u, rolled, p)
        packed.append(p[0] + offset)
    blk = jnp.concatenate(packed, axis=0)
    qbase = offset + rb * ROWS
    src = jax.lax.broadcasted_iota(jnp.int32, (ROWS, NEIGHBORS), 0) + qbase
    out_ref[0, 0] = src
    out_ref[1, 0] = blk.T


def kernel(h):
    n_samples, n_points, n_dims = h.shape
    grid = (n_samples, n_points // ROWS)
    edges = pl.pallas_call(
        _knn_kernel,
        grid=grid,
        in_specs=[
            pl.BlockSpec((1, ROWS, n_dims), lambda s, rb: (s, rb, 0)),
            pl.BlockSpec((1, n_points, n_dims), lambda s, rb: (s, 0, 0)),
        ],
        out_specs=pl.BlockSpec((2, 1, ROWS, NEIGHBORS),
                               lambda s, rb: (0, s, rb, 0)),
        out_shape=jax.ShapeDtypeStruct(
            (2, n_samples, n_points, NEIGHBORS), jnp.int32),
        scratch_shapes=[pltpu.VMEM((n_points, ROWS), jnp.float32)],
    )(h, h)
    return edges.reshape(2, -1)
